# baseline (speedup 1.0000x reference)
import functools

import jax
import jax.numpy as jnp
from jax import lax
from jax.experimental import pallas as pl
from jax.experimental.pallas import tpu as pltpu
from jax.experimental.pallas import tpu_sc as plsc

N_NODES = 10000
N_EDGES = 320000
D_FEAT = 128
POS_DIM = 4

D_PAD = 144
CHUNK = 128
NC = 2
NS = 16
NW = NC * NS
N_CHUNKS = 80
PER_W = N_CHUNKS * CHUNK
E_PAD = PER_W * NW
ROWS_PER_TILE = 640
ACC_ROWS = ROWS_PER_TILE * NS
PAD_ROW = N_NODES
XP_ROWS = 10048

_mesh = plsc.VectorSubcoreMesh(core_axis_name="c", subcore_axis_name="s")


NCH_C0 = 84
NCH_C1 = (NW * N_CHUNKS - NS * NCH_C0) // NS


@functools.partial(
    pl.kernel,
    out_type=jax.ShapeDtypeStruct((NC, ACC_ROWS, D_PAD), jnp.bfloat16),
    mesh=_mesh,
    scratch_types=[
        pltpu.VMEM((2, CHUNK), jnp.int32),
        pltpu.VMEM((2, CHUNK), jnp.int32),
        pltpu.VMEM((CHUNK, D_PAD), jnp.bfloat16),
        pltpu.VMEM((CHUNK, D_PAD), jnp.bfloat16),
        pltpu.VMEM_SHARED((ACC_ROWS, D_PAD), jnp.bfloat16),
        pltpu.VMEM_SHARED((XP_ROWS, D_PAD), jnp.bfloat16),
        pltpu.SemaphoreType.DMA,
        pltpu.SemaphoreType.DMA,
    ],
    compiler_params=pltpu.CompilerParams(use_tc_tiling_on_sc=False),
)
def _sc_scatter_accum(xp_hbm, edges_hbm, z_hbm, out_hbm,
                      idx_a, idx_b, rows_a, rows_b, acc, xp_spm, sem_a, sem_b):
    c = lax.axis_index("c")
    s = lax.axis_index("s")

    @pl.when(c == 1)
    def _stage():
        pltpu.sync_copy(
            xp_hbm.at[pl.ds(s * (XP_ROWS // NS), XP_ROWS // NS)],
            xp_spm.at[pl.ds(s * (XP_ROWS // NS), XP_ROWS // NS)])

    pltpu.sync_copy(z_hbm, rows_a)
    for b in range(ROWS_PER_TILE // CHUNK):
        pltpu.sync_copy(
            rows_a, acc.at[pl.ds(s * ROWS_PER_TILE + b * CHUNK, CHUNK)])
    plsc.subcore_barrier()

    def wait_gather(rows, sem):
        pltpu.make_async_copy(xp_hbm.at[pl.ds(0, CHUNK)], rows, sem).wait()

    def load_idx(ch, idx):
        pltpu.sync_copy(edges_hbm.at[0, ch], idx.at[0])
        pltpu.sync_copy(edges_hbm.at[1, ch], idx.at[1])

    def run_edges(table, cbase, nch):
        load_idx(cbase, idx_a)
        pltpu.async_copy(table.at[idx_a.at[0]], rows_a, sem_a)
        load_idx(cbase + 1, idx_b)
        pltpu.async_copy(table.at[idx_b.at[0]], rows_b, sem_b)

        def body(i, carry):
            ch = cbase + 2 * i
            wait_gather(rows_a, sem_a)
            pltpu.sync_copy(rows_a, acc.at[idx_a.at[1]], add=True)
            load_idx(ch + 2, idx_a)
            pltpu.async_copy(table.at[idx_a.at[0]], rows_a, sem_a)
            wait_gather(rows_b, sem_b)
            pltpu.sync_copy(rows_b, acc.at[idx_b.at[1]], add=True)
            load_idx(ch + 3, idx_b)
            pltpu.async_copy(table.at[idx_b.at[0]], rows_b, sem_b)
            return carry

        lax.fori_loop(0, nch // 2 - 1, body, 0)

        wait_gather(rows_a, sem_a)
        pltpu.sync_copy(rows_a, acc.at[idx_a.at[1]], add=True)
        wait_gather(rows_b, sem_b)
        pltpu.sync_copy(rows_b, acc.at[idx_b.at[1]], add=True)

    @pl.when(c == 0)
    def _run_c0():
        run_edges(xp_hbm, s * NCH_C0, NCH_C0)

    @pl.when(c == 1)
    def _run_c1():
        run_edges(xp_spm, NS * NCH_C0 + s * NCH_C1, NCH_C1)

    plsc.subcore_barrier()

    pltpu.sync_copy(
        acc.at[pl.ds(s * ROWS_PER_TILE, ROWS_PER_TILE)],
        out_hbm.at[c, pl.ds(s * ROWS_PER_TILE, ROWS_PER_TILE)],
    )


_R = 400


def _combine_body(part_ref, x_ref, pos_ref, out_ref):
    p = part_ref[0].astype(jnp.float32) + part_ref[1].astype(jnp.float32)
    deg = p[:, D_FEAT + POS_DIM:D_FEAT + POS_DIM + 1]
    outx = p[:, :D_FEAT] + x_ref[...]
    outp = p[:, D_FEAT:D_FEAT + POS_DIM] - deg * pos_ref[...]
    out_ref[...] = jnp.concatenate([outx, outp], axis=1)


_combine = pl.pallas_call(
    _combine_body,
    grid=(N_NODES // _R,),
    in_specs=[
        pl.BlockSpec((NC, _R, D_PAD), lambda i: (0, i, 0)),
        pl.BlockSpec((_R, D_FEAT), lambda i: (i, 0)),
        pl.BlockSpec((_R, POS_DIM), lambda i: (i, 0)),
    ],
    out_specs=pl.BlockSpec((_R, D_FEAT + POS_DIM), lambda i: (i, 0)),
    out_shape=jax.ShapeDtypeStruct((N_NODES, D_FEAT + POS_DIM), jnp.float32),
)


def kernel(x, pos, edge_index):
    ei = edge_index.astype(jnp.int32)
    pad = jnp.full((2, E_PAD - N_EDGES), PAD_ROW, jnp.int32)
    edges = jnp.concatenate([ei, pad], axis=1)
    edges = edges.reshape(2, E_PAD // CHUNK, CHUNK)
    xp = jnp.concatenate(
        [x, pos, jnp.ones((N_NODES, 1), jnp.float32)], axis=1)
    xp = jnp.pad(xp, ((0, XP_ROWS - N_NODES), (0, D_PAD - (D_FEAT + POS_DIM + 1))))
    xp = xp.astype(jnp.bfloat16)
    zeros_chunk = jnp.zeros((CHUNK, D_PAD), jnp.bfloat16)
    part = _sc_scatter_accum(xp, edges, zeros_chunk)
    return _combine(part, x, pos)


# speedup vs baseline: 1.1336x; 1.1336x over previous
import functools

import jax
import jax.numpy as jnp
from jax import lax
from jax.experimental import pallas as pl
from jax.experimental.pallas import tpu as pltpu
from jax.experimental.pallas import tpu_sc as plsc

N_NODES = 10000
N_EDGES = 320000
D_FEAT = 128
POS_DIM = 4

D_PAD = 144
CHUNK = 128
NC = 2
NS = 16
NW = NC * NS
N_CHUNKS = 80
PER_W = N_CHUNKS * CHUNK
E_PAD = PER_W * NW
ROWS_PER_TILE = 640
ACC_ROWS = ROWS_PER_TILE * NS
PAD_ROW = N_NODES
XP_ROWS = 10048

_mesh = plsc.VectorSubcoreMesh(core_axis_name="c", subcore_axis_name="s")


NCH_C0 = 84
NCH_C1 = (NW * N_CHUNKS - NS * NCH_C0) // NS


@functools.partial(
    pl.kernel,
    out_type=jax.ShapeDtypeStruct((NC, ACC_ROWS, D_PAD), jnp.bfloat16),
    mesh=_mesh,
    scratch_types=[
        pltpu.VMEM((2, CHUNK), jnp.int32),
        pltpu.VMEM((2, CHUNK), jnp.int32),
        pltpu.VMEM((CHUNK, D_PAD), jnp.bfloat16),
        pltpu.VMEM((CHUNK, D_PAD), jnp.bfloat16),
        pltpu.VMEM_SHARED((ACC_ROWS, D_PAD), jnp.bfloat16),
        pltpu.VMEM_SHARED((XP_ROWS, D_PAD), jnp.bfloat16),
        pltpu.SemaphoreType.DMA,
        pltpu.SemaphoreType.DMA,
    ],
    compiler_params=pltpu.CompilerParams(use_tc_tiling_on_sc=False),
)
def _sc_scatter_accum(xp_hbm, edges_hbm, z_hbm, out_hbm,
                      idx_a, idx_b, rows_a, rows_b, acc, xp_spm, sem_a, sem_b):
    c = lax.axis_index("c")
    s = lax.axis_index("s")

    @pl.when(c == 1)
    def _stage():
        pltpu.sync_copy(
            xp_hbm.at[pl.ds(s * (XP_ROWS // NS), XP_ROWS // NS)],
            xp_spm.at[pl.ds(s * (XP_ROWS // NS), XP_ROWS // NS)])

    pltpu.sync_copy(z_hbm, rows_a)
    for b in range(ROWS_PER_TILE // CHUNK):
        pltpu.sync_copy(
            rows_a, acc.at[pl.ds(s * ROWS_PER_TILE + b * CHUNK, CHUNK)])
    plsc.subcore_barrier()

    def wait_gather(rows, sem):
        pltpu.make_async_copy(xp_hbm.at[pl.ds(0, CHUNK)], rows, sem).wait()

    def load_idx(ch, idx):
        pltpu.sync_copy(edges_hbm.at[ch], idx)

    def run_edges(table, cbase, nch):
        load_idx(cbase, idx_a)
        pltpu.async_copy(table.at[idx_a.at[0]], rows_a, sem_a)
        load_idx(cbase + 1, idx_b)
        pltpu.async_copy(table.at[idx_b.at[0]], rows_b, sem_b)

        def body(i, carry):
            ch = cbase + 2 * i
            wait_gather(rows_a, sem_a)
            pltpu.sync_copy(rows_a, acc.at[idx_a.at[1]], add=True)
            load_idx(ch + 2, idx_a)
            pltpu.async_copy(table.at[idx_a.at[0]], rows_a, sem_a)
            wait_gather(rows_b, sem_b)
            pltpu.sync_copy(rows_b, acc.at[idx_b.at[1]], add=True)
            load_idx(ch + 3, idx_b)
            pltpu.async_copy(table.at[idx_b.at[0]], rows_b, sem_b)
            return carry

        lax.fori_loop(0, nch // 2 - 1, body, 0)

        wait_gather(rows_a, sem_a)
        pltpu.sync_copy(rows_a, acc.at[idx_a.at[1]], add=True)
        wait_gather(rows_b, sem_b)
        pltpu.sync_copy(rows_b, acc.at[idx_b.at[1]], add=True)

    @pl.when(c == 0)
    def _run_c0():
        run_edges(xp_hbm, s * NCH_C0, NCH_C0)

    @pl.when(c == 1)
    def _run_c1():
        run_edges(xp_spm, NS * NCH_C0 + s * NCH_C1, NCH_C1)

    plsc.subcore_barrier()

    pltpu.sync_copy(
        acc.at[pl.ds(s * ROWS_PER_TILE, ROWS_PER_TILE)],
        out_hbm.at[c, pl.ds(s * ROWS_PER_TILE, ROWS_PER_TILE)],
    )


_R = 400


def _combine_body(part_ref, x_ref, pos_ref, out_ref):
    p = part_ref[0].astype(jnp.float32) + part_ref[1].astype(jnp.float32)
    deg = p[:, D_FEAT + POS_DIM:D_FEAT + POS_DIM + 1]
    outx = p[:, :D_FEAT] + x_ref[...]
    outp = p[:, D_FEAT:D_FEAT + POS_DIM] - deg * pos_ref[...]
    out_ref[...] = jnp.concatenate([outx, outp], axis=1)


_combine = pl.pallas_call(
    _combine_body,
    grid=(N_NODES // _R,),
    in_specs=[
        pl.BlockSpec((NC, _R, D_PAD), lambda i: (0, i, 0)),
        pl.BlockSpec((_R, D_FEAT), lambda i: (i, 0)),
        pl.BlockSpec((_R, POS_DIM), lambda i: (i, 0)),
    ],
    out_specs=pl.BlockSpec((_R, D_FEAT + POS_DIM), lambda i: (i, 0)),
    out_shape=jax.ShapeDtypeStruct((N_NODES, D_FEAT + POS_DIM), jnp.float32),
)


def kernel(x, pos, edge_index):
    ei = edge_index.astype(jnp.int32)
    pad = jnp.full((2, E_PAD - N_EDGES), PAD_ROW, jnp.int32)
    edges = jnp.concatenate([ei, pad], axis=1)
    edges = edges.reshape(2, E_PAD // CHUNK, CHUNK).transpose(1, 0, 2)
    xp = jnp.concatenate(
        [x, pos, jnp.ones((N_NODES, 1), jnp.float32)], axis=1)
    xp = jnp.pad(xp, ((0, XP_ROWS - N_NODES), (0, D_PAD - (D_FEAT + POS_DIM + 1))))
    xp = xp.astype(jnp.bfloat16)
    zeros_chunk = jnp.zeros((CHUNK, D_PAD), jnp.bfloat16)
    part = _sc_scatter_accum(xp, edges, zeros_chunk)
    return _combine(part, x, pos)


# speedup vs baseline: 1.1747x; 1.0363x over previous
import functools

import jax
import jax.numpy as jnp
from jax import lax
from jax.experimental import pallas as pl
from jax.experimental.pallas import tpu as pltpu
from jax.experimental.pallas import tpu_sc as plsc

N_NODES = 10000
N_EDGES = 320000
D_FEAT = 128
POS_DIM = 4

D_PAD = 144
CHUNK = 128
NC = 2
NS = 16
NW = NC * NS
N_CHUNKS = 80
PER_W = N_CHUNKS * CHUNK
E_PAD = PER_W * NW
ROWS_PER_TILE = 640
ACC_ROWS = ROWS_PER_TILE * NS
PAD_ROW = N_NODES
XP_ROWS = 10048

_mesh = plsc.VectorSubcoreMesh(core_axis_name="c", subcore_axis_name="s")


NCH_C0 = 92
NCH_C1 = (NW * N_CHUNKS - NS * NCH_C0) // NS


@functools.partial(
    pl.kernel,
    out_type=jax.ShapeDtypeStruct((NC, ACC_ROWS, D_PAD), jnp.bfloat16),
    mesh=_mesh,
    scratch_types=[
        pltpu.VMEM((2, CHUNK), jnp.int32),
        pltpu.VMEM((2, CHUNK), jnp.int32),
        pltpu.VMEM((CHUNK, D_PAD), jnp.bfloat16),
        pltpu.VMEM((CHUNK, D_PAD), jnp.bfloat16),
        pltpu.VMEM_SHARED((ACC_ROWS, D_PAD), jnp.bfloat16),
        pltpu.VMEM_SHARED((XP_ROWS, D_PAD), jnp.bfloat16),
        pltpu.SemaphoreType.DMA,
        pltpu.SemaphoreType.DMA,
    ],
    compiler_params=pltpu.CompilerParams(use_tc_tiling_on_sc=False),
)
def _sc_scatter_accum(xp_hbm, edges_hbm, z_hbm, out_hbm,
                      idx_a, idx_b, rows_a, rows_b, acc, xp_spm, sem_a, sem_b):
    c = lax.axis_index("c")
    s = lax.axis_index("s")

    @pl.when(c == 1)
    def _stage():
        pltpu.sync_copy(
            xp_hbm.at[pl.ds(s * (XP_ROWS // NS), XP_ROWS // NS)],
            xp_spm.at[pl.ds(s * (XP_ROWS // NS), XP_ROWS // NS)])

    pltpu.sync_copy(z_hbm, rows_a)
    for b in range(ROWS_PER_TILE // CHUNK):
        pltpu.sync_copy(
            rows_a, acc.at[pl.ds(s * ROWS_PER_TILE + b * CHUNK, CHUNK)])
    plsc.subcore_barrier()

    def wait_gather(rows, sem):
        pltpu.make_async_copy(xp_hbm.at[pl.ds(0, CHUNK)], rows, sem).wait()

    def load_idx(ch, idx):
        pltpu.sync_copy(edges_hbm.at[ch], idx)

    def run_edges(table, cbase, nch):
        load_idx(cbase, idx_a)
        pltpu.async_copy(table.at[idx_a.at[0]], rows_a, sem_a)
        load_idx(cbase + 1, idx_b)
        pltpu.async_copy(table.at[idx_b.at[0]], rows_b, sem_b)

        def body(i, carry):
            ch = cbase + 2 * i
            wait_gather(rows_a, sem_a)
            pltpu.sync_copy(rows_a, acc.at[idx_a.at[1]], add=True)
            load_idx(ch + 2, idx_a)
            pltpu.async_copy(table.at[idx_a.at[0]], rows_a, sem_a)
            wait_gather(rows_b, sem_b)
            pltpu.sync_copy(rows_b, acc.at[idx_b.at[1]], add=True)
            load_idx(ch + 3, idx_b)
            pltpu.async_copy(table.at[idx_b.at[0]], rows_b, sem_b)
            return carry

        lax.fori_loop(0, nch // 2 - 1, body, 0)

        wait_gather(rows_a, sem_a)
        pltpu.sync_copy(rows_a, acc.at[idx_a.at[1]], add=True)
        wait_gather(rows_b, sem_b)
        pltpu.sync_copy(rows_b, acc.at[idx_b.at[1]], add=True)

    @pl.when(c == 0)
    def _run_c0():
        run_edges(xp_hbm, s * NCH_C0, NCH_C0)

    @pl.when(c == 1)
    def _run_c1():
        run_edges(xp_spm, NS * NCH_C0 + s * NCH_C1, NCH_C1)

    plsc.subcore_barrier()

    pltpu.sync_copy(
        acc.at[pl.ds(s * ROWS_PER_TILE, ROWS_PER_TILE)],
        out_hbm.at[c, pl.ds(s * ROWS_PER_TILE, ROWS_PER_TILE)],
    )


_R = 400


def _combine_body(part_ref, x_ref, pos_ref, out_ref):
    p = part_ref[0].astype(jnp.float32) + part_ref[1].astype(jnp.float32)
    deg = p[:, D_FEAT + POS_DIM:D_FEAT + POS_DIM + 1]
    outx = p[:, :D_FEAT] + x_ref[...]
    outp = p[:, D_FEAT:D_FEAT + POS_DIM] - deg * pos_ref[...]
    out_ref[...] = jnp.concatenate([outx, outp], axis=1)


_combine = pl.pallas_call(
    _combine_body,
    grid=(N_NODES // _R,),
    in_specs=[
        pl.BlockSpec((NC, _R, D_PAD), lambda i: (0, i, 0)),
        pl.BlockSpec((_R, D_FEAT), lambda i: (i, 0)),
        pl.BlockSpec((_R, POS_DIM), lambda i: (i, 0)),
    ],
    out_specs=pl.BlockSpec((_R, D_FEAT + POS_DIM), lambda i: (i, 0)),
    out_shape=jax.ShapeDtypeStruct((N_NODES, D_FEAT + POS_DIM), jnp.float32),
)


def kernel(x, pos, edge_index):
    ei = edge_index.astype(jnp.int32)
    pad = jnp.full((2, E_PAD - N_EDGES), PAD_ROW, jnp.int32)
    edges = jnp.concatenate([ei, pad], axis=1)
    edges = edges.reshape(2, E_PAD // CHUNK, CHUNK).transpose(1, 0, 2)
    xp = jnp.concatenate(
        [x, pos, jnp.ones((N_NODES, 1), jnp.float32)], axis=1)
    xp = jnp.pad(xp, ((0, XP_ROWS - N_NODES), (0, D_PAD - (D_FEAT + POS_DIM + 1))))
    xp = xp.astype(jnp.bfloat16)
    zeros_chunk = jnp.zeros((CHUNK, D_PAD), jnp.bfloat16)
    part = _sc_scatter_accum(xp, edges, zeros_chunk)
    return _combine(part, x, pos)
